# baseline (speedup 1.0000x reference)
import jax
import jax.numpy as jnp
from jax import lax
from jax.experimental import pallas as pl
from jax.experimental.pallas import tpu as pltpu
from jax.experimental.pallas import tpu_sc as plsc

_F = 64
_CAV = 5
_NX, _NY = 504, 200
_NP = _NY * _NX
_TOT = _CAV * _NP
_P = 40000
_PTAB = _P + 64
_NSC = 2
_NTILE = 16
_PPT = 2560
_PPAD = _NTILE * _PPT
_PC = 256
_K = 2240
_NCH = _NP // _K
_UNITS = _CAV * _NCH
_SENT = _P
_INVSZ = _TOT + 16


def _tr_body(x_ref, o_ref):
    i = pl.program_id(0)
    col = jax.lax.broadcasted_iota(jnp.int32, (64, 128), 1) + i * 128
    o_ref[...] = jnp.where(col < _P, x_ref[...].T, 0.0)


def _feature_table(feat):
    return pl.pallas_call(
        _tr_body,
        grid=(_PTAB // 128,),
        in_specs=[pl.BlockSpec((128, 64), lambda i: (i, 0))],
        out_specs=pl.BlockSpec((64, 128), lambda i: (0, i)),
        out_shape=jax.ShapeDtypeStruct((64, _PTAB), jnp.float32),
    )(feat)


def _sc_body(cb_hbm, cy_hbm, cx_hbm, tab_hbm, out_hbm, inv_hbm,
             inv_sh, tab_v, cb_v, cy_v, cx_v, lin_v, pid_v,
             invc0_v, invc1_v, oba0_v, oba1_v, obb0_v, obb1_v,
             si0, si1, sa0, sa1, sb0, sb1):
    cid = lax.axis_index("c")
    sid = lax.axis_index("s")
    w = cid * _NTILE + sid
    iot = lax.iota(jnp.int32, 16)

    def _fill(i, c):
        invc0_v[pl.ds(i * 16, 16)] = jnp.full((16,), _SENT, jnp.int32)
        return c

    lax.fori_loop(0, _K // 16, _fill, 0)
    for k in range((_TOT // _K + _NTILE - 1) // _NTILE):
        ch = sid + _NTILE * k

        @pl.when(ch < _TOT // _K)
        def _():
            pltpu.sync_copy(invc0_v, inv_sh.at[pl.ds(ch * _K, _K)])
    plsc.subcore_barrier()

    base = sid * _PPT
    for q in range(_PPT // _PC):
        qb = base + q * _PC
        pltpu.sync_copy(cb_hbm.at[pl.ds(qb, _PC)], cb_v)
        pltpu.sync_copy(cy_hbm.at[pl.ds(qb, _PC)], cy_v)
        pltpu.sync_copy(cx_hbm.at[pl.ds(qb, _PC)], cx_v)
        for j in range(_PC // 128):
            def _row(l, c, j=j, qb=qb):
                s = pl.ds(j * 128 + l * 16, 16)
                lin_v[j, pl.ds(l * 16, 16)] = (
                    cb_v[s] * _NP + cy_v[s] * _NX + cx_v[s])
                pid_v[j, pl.ds(l * 16, 16)] = qb + j * 128 + l * 16 + iot
                return c

            lax.fori_loop(0, 8, _row, 0)
            pltpu.sync_copy(pid_v.at[j], inv_sh.at[lin_v.at[j]])
    plsc.subcore_barrier()

    invbase = cid * _INVSZ
    for k in range((_TOT // _K + _NTILE - 1) // _NTILE):
        ch = sid + _NTILE * k

        @pl.when(ch < _TOT // _K)
        def _():
            pltpu.sync_copy(inv_sh.at[pl.ds(ch * _K, _K)], invc0_v)
            pltpu.sync_copy(invc0_v, inv_hbm.at[pl.ds(invbase + ch * _K, _K)])
    plsc.subcore_barrier()

    pltpu.sync_copy(tab_hbm.at[pl.ds(2 * w * _PTAB, 2 * _PTAB)], tab_v)

    def _in_slice(u):
        return inv_hbm.at[
            pl.ds(invbase + (u // _NCH) * _NP + (u % _NCH) * _K, _K)]

    def _obase(u):
        return ((u // _NCH) * _F + 2 * w) * _NP + (u % _NCH) * _K

    def _gpass(invbuf, obbuf, off):
        def _g(i, cc):
            for r in range(4):
                s = pl.ds(i * 64 + r * 16, 16)
                obbuf[s] = plsc.load_gather(tab_v, [invbuf[s] + off])
            return cc

        lax.fori_loop(0, _K // 64, _g, 0)

    invc = (invc0_v, invc1_v)
    oba = (oba0_v, oba1_v)
    obb = (obb0_v, obb1_v)
    si = (si0, si1)
    sa = (sa0, sa1)
    sb = (sb0, sb1)

    def _wait_out(u, par):
        pltpu.make_async_copy(
            oba[par], out_hbm.at[pl.ds(_obase(u), _K)], sa[par]).wait()
        pltpu.make_async_copy(
            obb[par], out_hbm.at[pl.ds(_obase(u) + _NP, _K)], sb[par]).wait()

    def _do_unit(u, par):
        pltpu.make_async_copy(_in_slice(u), invc[par], si[par]).wait()

        @pl.when(u + 1 < _UNITS)
        def _():
            pltpu.async_copy(_in_slice(u + 1), invc[1 - par], si[1 - par])

        @pl.when(u >= 2)
        def _():
            _wait_out(u - 2, par)

        _gpass(invc[par], oba[par], 0)
        _gpass(invc[par], obb[par], _PTAB)
        pltpu.async_copy(oba[par], out_hbm.at[pl.ds(_obase(u), _K)], sa[par])
        pltpu.async_copy(
            obb[par], out_hbm.at[pl.ds(_obase(u) + _NP, _K)], sb[par])

    pltpu.async_copy(_in_slice(0), invc[0], si[0])

    def _outer(t, c):
        for par in (0, 1):
            _do_unit(2 * t + par, par)
        return c

    lax.fori_loop(0, _UNITS // 2, _outer, 0)
    _do_unit(_UNITS - 1, (_UNITS - 1) % 2)
    _wait_out(_UNITS - 2, (_UNITS - 2) % 2)
    _wait_out(_UNITS - 1, (_UNITS - 1) % 2)


_sc_scatter = pl.kernel(
    _sc_body,
    out_type=(
        jax.ShapeDtypeStruct((_CAV * _F * _NP,), jnp.float32),
        jax.ShapeDtypeStruct((_NSC * _INVSZ,), jnp.int32),
    ),
    mesh=plsc.VectorSubcoreMesh(core_axis_name="c", subcore_axis_name="s"),
    compiler_params=pltpu.CompilerParams(needs_layout_passes=False),
    scratch_types=[
        pltpu.VMEM_SHARED((_INVSZ,), jnp.int32),
        pltpu.VMEM((2 * _PTAB,), jnp.float32),
        pltpu.VMEM((_PC,), jnp.int32),
        pltpu.VMEM((_PC,), jnp.int32),
        pltpu.VMEM((_PC,), jnp.int32),
        pltpu.VMEM((_PC // 128, 128), jnp.int32),
        pltpu.VMEM((_PC // 128, 128), jnp.int32),
        pltpu.VMEM((_K,), jnp.int32),
        pltpu.VMEM((_K,), jnp.int32),
        pltpu.VMEM((_K,), jnp.float32),
        pltpu.VMEM((_K,), jnp.float32),
        pltpu.VMEM((_K,), jnp.float32),
        pltpu.VMEM((_K,), jnp.float32),
        pltpu.SemaphoreType.DMA,
        pltpu.SemaphoreType.DMA,
        pltpu.SemaphoreType.DMA,
        pltpu.SemaphoreType.DMA,
        pltpu.SemaphoreType.DMA,
        pltpu.SemaphoreType.DMA,
    ],
)


def kernel(voxel_coords, record_len, pillar_features):
    del record_len
    cb = voxel_coords[:, 0].astype(jnp.int32)
    cy = voxel_coords[:, 2].astype(jnp.int32)
    cx = voxel_coords[:, 3].astype(jnp.int32)
    pad = _PPAD - cb.shape[0]
    cb = jnp.concatenate([cb, jnp.full((pad,), _CAV, jnp.int32)])
    cy = jnp.concatenate([cy, jnp.zeros((pad,), jnp.int32)])
    cx = jnp.concatenate([cx, jnp.zeros((pad,), jnp.int32)])
    tab = _feature_table(pillar_features).reshape(_F * _PTAB)
    out, _ = _sc_scatter(cb, cy, cx, tab)
    return out.reshape(_CAV, _F, _NY, _NX)


# speedup vs baseline: 1.1565x; 1.1565x over previous
import jax
import jax.numpy as jnp
from jax import lax
from jax.experimental import pallas as pl
from jax.experimental.pallas import tpu as pltpu
from jax.experimental.pallas import tpu_sc as plsc

_F = 64
_CAV = 5
_NX, _NY = 504, 200
_NP = _NY * _NX
_TOT = _CAV * _NP
_P = 40000
_PTAB = _P + 64
_NSC = 2
_NTILE = 16
_PPT = 2560
_PPAD = _NTILE * _PPT
_PC = 256
_K = 2240
_NCH = _NP // _K
_UNITS = _CAV * _NCH
_SENT = _P
_INVSZ = _TOT + 16


def _tr_body(x_ref, o_ref):
    i = pl.program_id(0)
    col = jax.lax.broadcasted_iota(jnp.int32, (64, 128), 1) + i * 128
    o_ref[...] = jnp.where(col < _P, x_ref[...].T, 0.0)


def _feature_table(feat):
    return pl.pallas_call(
        _tr_body,
        grid=(_PTAB // 128,),
        in_specs=[pl.BlockSpec((128, 64), lambda i: (i, 0))],
        out_specs=pl.BlockSpec((64, 128), lambda i: (0, i)),
        out_shape=jax.ShapeDtypeStruct((64, _PTAB), jnp.float32),
    )(feat)


def _sc_body(cb_hbm, cy_hbm, cx_hbm, tab_hbm, out_hbm, inv_hbm,
             inv_sh, tab_v, cb_v, cy_v, cx_v, lin_v, pid_v,
             invc0_v, invc1_v, oba0_v, oba1_v, obb0_v, obb1_v,
             si0, si1, sa0, sa1, sb0, sb1):
    cid = lax.axis_index("c")
    sid = lax.axis_index("s")
    w = cid * _NTILE + sid
    iot = lax.iota(jnp.int32, 16)

    def _fill(i, c):
        invc0_v[pl.ds(i * 16, 16)] = jnp.full((16,), _SENT, jnp.int32)
        return c

    lax.fori_loop(0, _K // 16, _fill, 0)
    for k in range((_TOT // _K + _NTILE - 1) // _NTILE):
        ch = sid + _NTILE * k

        @pl.when(ch < _TOT // _K)
        def _():
            pltpu.sync_copy(invc0_v, inv_sh.at[pl.ds(ch * _K, _K)])
    plsc.subcore_barrier()

    base = sid * _PPT
    for q in range(_PPT // _PC):
        qb = base + q * _PC
        pltpu.sync_copy(cb_hbm.at[pl.ds(qb, _PC)], cb_v)
        pltpu.sync_copy(cy_hbm.at[pl.ds(qb, _PC)], cy_v)
        pltpu.sync_copy(cx_hbm.at[pl.ds(qb, _PC)], cx_v)
        for j in range(_PC // 128):
            def _row(l, c, j=j, qb=qb):
                s = pl.ds(j * 128 + l * 16, 16)
                lin_v[j, pl.ds(l * 16, 16)] = (
                    cb_v[s] * _NP + cy_v[s] * _NX + cx_v[s])
                pid_v[j, pl.ds(l * 16, 16)] = qb + j * 128 + l * 16 + iot
                return c

            lax.fori_loop(0, 8, _row, 0)
            pltpu.sync_copy(pid_v.at[j], inv_sh.at[lin_v.at[j]])
    plsc.subcore_barrier()

    invbase = cid * _INVSZ
    for k in range((_TOT // _K + _NTILE - 1) // _NTILE):
        ch = sid + _NTILE * k

        @pl.when(ch < _TOT // _K)
        def _():
            pltpu.sync_copy(inv_sh.at[pl.ds(ch * _K, _K)], invc0_v)
            pltpu.sync_copy(invc0_v, inv_hbm.at[pl.ds(invbase + ch * _K, _K)])
    plsc.subcore_barrier()

    pltpu.sync_copy(tab_hbm.at[pl.ds(2 * w * _PTAB, 2 * _PTAB)], tab_v)

    def _in_slice(u):
        return inv_hbm.at[
            pl.ds(invbase + (u // _NCH) * _NP + (u % _NCH) * _K, _K)]

    def _obase(u):
        return ((u // _NCH) * _F + 2 * w) * _NP + (u % _NCH) * _K

    def _gfused(invbuf, ob0, ob1):
        @plsc.parallel_loop(0, _K // 64, unroll=2)
        def _g(i):
            for r in range(4):
                s = pl.ds(i * 64 + r * 16, 16)
                inv = invbuf[s]
                ob0[s] = plsc.load_gather(tab_v, [inv])
                ob1[s] = plsc.load_gather(tab_v, [inv + _PTAB])

    invc = (invc0_v, invc1_v)
    oba = (oba0_v, oba1_v)
    obb = (obb0_v, obb1_v)
    si = (si0, si1)
    sa = (sa0, sa1)
    sb = (sb0, sb1)

    def _wait_out(u, par):
        pltpu.make_async_copy(
            oba[par], out_hbm.at[pl.ds(_obase(u), _K)], sa[par]).wait()
        pltpu.make_async_copy(
            obb[par], out_hbm.at[pl.ds(_obase(u) + _NP, _K)], sb[par]).wait()

    def _do_unit(u, par):
        pltpu.make_async_copy(_in_slice(u), invc[par], si[par]).wait()

        @pl.when(u + 1 < _UNITS)
        def _():
            pltpu.async_copy(_in_slice(u + 1), invc[1 - par], si[1 - par])

        @pl.when(u >= 2)
        def _():
            _wait_out(u - 2, par)

        _gfused(invc[par], oba[par], obb[par])
        pltpu.async_copy(oba[par], out_hbm.at[pl.ds(_obase(u), _K)], sa[par])
        pltpu.async_copy(
            obb[par], out_hbm.at[pl.ds(_obase(u) + _NP, _K)], sb[par])

    pltpu.async_copy(_in_slice(0), invc[0], si[0])

    def _outer(t, c):
        for par in (0, 1):
            _do_unit(2 * t + par, par)
        return c

    lax.fori_loop(0, _UNITS // 2, _outer, 0)
    _do_unit(_UNITS - 1, (_UNITS - 1) % 2)
    _wait_out(_UNITS - 2, (_UNITS - 2) % 2)
    _wait_out(_UNITS - 1, (_UNITS - 1) % 2)


_sc_scatter = pl.kernel(
    _sc_body,
    out_type=(
        jax.ShapeDtypeStruct((_CAV * _F * _NP,), jnp.float32),
        jax.ShapeDtypeStruct((_NSC * _INVSZ,), jnp.int32),
    ),
    mesh=plsc.VectorSubcoreMesh(core_axis_name="c", subcore_axis_name="s"),
    compiler_params=pltpu.CompilerParams(needs_layout_passes=False),
    scratch_types=[
        pltpu.VMEM_SHARED((_INVSZ,), jnp.int32),
        pltpu.VMEM((2 * _PTAB,), jnp.float32),
        pltpu.VMEM((_PC,), jnp.int32),
        pltpu.VMEM((_PC,), jnp.int32),
        pltpu.VMEM((_PC,), jnp.int32),
        pltpu.VMEM((_PC // 128, 128), jnp.int32),
        pltpu.VMEM((_PC // 128, 128), jnp.int32),
        pltpu.VMEM((_K,), jnp.int32),
        pltpu.VMEM((_K,), jnp.int32),
        pltpu.VMEM((_K,), jnp.float32),
        pltpu.VMEM((_K,), jnp.float32),
        pltpu.VMEM((_K,), jnp.float32),
        pltpu.VMEM((_K,), jnp.float32),
        pltpu.SemaphoreType.DMA,
        pltpu.SemaphoreType.DMA,
        pltpu.SemaphoreType.DMA,
        pltpu.SemaphoreType.DMA,
        pltpu.SemaphoreType.DMA,
        pltpu.SemaphoreType.DMA,
    ],
)


def kernel(voxel_coords, record_len, pillar_features):
    del record_len
    cb = voxel_coords[:, 0].astype(jnp.int32)
    cy = voxel_coords[:, 2].astype(jnp.int32)
    cx = voxel_coords[:, 3].astype(jnp.int32)
    pad = _PPAD - cb.shape[0]
    cb = jnp.concatenate([cb, jnp.full((pad,), _CAV, jnp.int32)])
    cy = jnp.concatenate([cy, jnp.zeros((pad,), jnp.int32)])
    cx = jnp.concatenate([cx, jnp.zeros((pad,), jnp.int32)])
    tab = _feature_table(pillar_features).reshape(_F * _PTAB)
    out, _ = _sc_scatter(cb, cy, cx, tab)
    return out.reshape(_CAV, _F, _NY, _NX)


# speedup vs baseline: 2.0884x; 1.8058x over previous
import jax
import jax.numpy as jnp
from jax import lax
from jax.experimental import pallas as pl
from jax.experimental.pallas import tpu as pltpu
from jax.experimental.pallas import tpu_sc as plsc

_F = 64
_CAV = 5
_NX, _NY = 504, 200
_NP = _NY * _NX
_TOT = _CAV * _NP
_P = 40000
_PTAB = 40960
_NSC = 2
_NTILE = 16
_PPT = 2560
_PPAD = _NTILE * _PPT
_NROW = _PPT // 128
_ROWS = 8
_K = _ROWS * _NX
_NCH = _NP // _K
_UNITS = _CAV * _NCH
_SENT = _P
_PLSZ = _NP + 16


def _tr_body(x_ref, o_ref):
    i = pl.program_id(0)
    col = jax.lax.broadcasted_iota(jnp.int32, (64, 2048), 1) + i * 2048
    o_ref[...] = jnp.where(col < _P, x_ref[...].T, 0.0)


def _feature_table(feat):
    return pl.pallas_call(
        _tr_body,
        grid=(_PTAB // 2048,),
        in_specs=[pl.BlockSpec((2048, 64), lambda i: (i, 0))],
        out_specs=pl.BlockSpec((64, 2048), lambda i: (0, i)),
        out_shape=jax.ShapeDtypeStruct((64, _PTAB), jnp.float32),
    )(feat)


def _sc_body(cb_hbm, cy_hbm, cx_hbm, tab_hbm, out_hbm, inv_hbm,
             inv_sh, tab_v, cb_v, cy_v, cx_v, lin_v, pid_v,
             invc0_v, invc1_v, oba0_v, oba1_v, obb0_v, obb1_v,
             si0, si1, sa0, sa1, sb0, sb1):
    cid = lax.axis_index("c")
    sid = lax.axis_index("s")
    w = cid * _NTILE + sid
    iot = lax.iota(jnp.int32, 16)
    invbase = cid * _TOT

    def _fill(i, c):
        invc0_v[pl.ds(i * 16, 16)] = jnp.full((16,), _SENT, jnp.int32)
        return c

    lax.fori_loop(0, _K // 16, _fill, 0)

    base = sid * _PPT
    pltpu.sync_copy(cb_hbm.at[pl.ds(base, _PPT)], cb_v)
    pltpu.sync_copy(cy_hbm.at[pl.ds(base, _PPT)], cy_v)
    pltpu.sync_copy(cx_hbm.at[pl.ds(base, _PPT)], cx_v)

    for b in range(_CAV):
        for k in range((_NCH + _NTILE - 1) // _NTILE):
            ch = sid + _NTILE * k

            @pl.when(ch < _NCH)
            def _():
                pltpu.sync_copy(invc0_v, inv_sh.at[pl.ds(ch * _K, _K)])
        plsc.subcore_barrier()

        for j in range(_NROW):
            def _row(l, c, j=j, b=b):
                s = pl.ds(j * 128 + l * 16, 16)
                lin = cy_v[s] * _NX + cx_v[s]
                lin_v[j, pl.ds(l * 16, 16)] = jnp.where(cb_v[s] == b, lin, _NP)
                pid_v[j, pl.ds(l * 16, 16)] = base + j * 128 + l * 16 + iot
                return c

            lax.fori_loop(0, 8, _row, 0)
            pltpu.sync_copy(pid_v.at[j], inv_sh.at[lin_v.at[j]])
        plsc.subcore_barrier()

        for k in range((_NCH + _NTILE - 1) // _NTILE):
            ch = sid + _NTILE * k

            @pl.when(ch < _NCH)
            def _():
                pltpu.sync_copy(inv_sh.at[pl.ds(ch * _K, _K)], invc1_v)
                pltpu.sync_copy(
                    invc1_v, inv_hbm.at[pl.ds(invbase + b * _NP + ch * _K, _K)])
        plsc.subcore_barrier()

    pltpu.sync_copy(tab_hbm.at[pl.ds(2 * w * _PTAB, 2 * _PTAB)], tab_v)

    def _in_slice(u):
        return inv_hbm.at[
            pl.ds(invbase + (u // _NCH) * _NP + (u % _NCH) * _K, _K)]

    def _out_slice(u, c):
        return out_hbm.at[u // _NCH, c, pl.ds((u % _NCH) * _ROWS, _ROWS)]

    def _gfused(invbuf, ob0, ob1):
        @plsc.parallel_loop(0, _K // 64, unroll=2)
        def _g(i):
            for r in range(4):
                pos = i * 64 + r * 16
                s = pl.ds(pos, 16)
                row = (pos + iot) // _NX
                col = (pos + iot) - row * _NX
                inv = invbuf[s]
                plsc.store_scatter(
                    ob0, [row, col], plsc.load_gather(tab_v, [inv]))
                plsc.store_scatter(
                    ob1, [row, col], plsc.load_gather(tab_v, [inv + _PTAB]))

    invc = (invc0_v, invc1_v)
    oba = (oba0_v, oba1_v)
    obb = (obb0_v, obb1_v)
    si = (si0, si1)
    sa = (sa0, sa1)
    sb = (sb0, sb1)

    def _wait_out(u, par):
        pltpu.make_async_copy(oba[par], _out_slice(u, 2 * w), sa[par]).wait()
        pltpu.make_async_copy(
            obb[par], _out_slice(u, 2 * w + 1), sb[par]).wait()

    def _do_unit(u, par):
        pltpu.make_async_copy(_in_slice(u), invc[par], si[par]).wait()

        @pl.when(u + 1 < _UNITS)
        def _():
            pltpu.async_copy(_in_slice(u + 1), invc[1 - par], si[1 - par])

        @pl.when(u >= 2)
        def _():
            _wait_out(u - 2, par)

        _gfused(invc[par], oba[par], obb[par])
        pltpu.async_copy(oba[par], _out_slice(u, 2 * w), sa[par])
        pltpu.async_copy(obb[par], _out_slice(u, 2 * w + 1), sb[par])

    pltpu.async_copy(_in_slice(0), invc[0], si[0])

    def _outer(t, c):
        for par in (0, 1):
            _do_unit(2 * t + par, par)
        return c

    lax.fori_loop(0, _UNITS // 2, _outer, 0)
    _do_unit(_UNITS - 1, (_UNITS - 1) % 2)
    _wait_out(_UNITS - 2, (_UNITS - 2) % 2)
    _wait_out(_UNITS - 1, (_UNITS - 1) % 2)


_sc_scatter = pl.kernel(
    _sc_body,
    out_type=(
        jax.ShapeDtypeStruct((_CAV, _F, _NY, _NX), jnp.float32),
        jax.ShapeDtypeStruct((_NSC * _TOT,), jnp.int32),
    ),
    mesh=plsc.VectorSubcoreMesh(core_axis_name="c", subcore_axis_name="s"),
    compiler_params=pltpu.CompilerParams(needs_layout_passes=False),
    scratch_types=[
        pltpu.VMEM_SHARED((_PLSZ,), jnp.int32),
        pltpu.VMEM((2 * _PTAB,), jnp.float32),
        pltpu.VMEM((_PPT,), jnp.int32),
        pltpu.VMEM((_PPT,), jnp.int32),
        pltpu.VMEM((_PPT,), jnp.int32),
        pltpu.VMEM((_NROW, 128), jnp.int32),
        pltpu.VMEM((_NROW, 128), jnp.int32),
        pltpu.VMEM((_K,), jnp.int32),
        pltpu.VMEM((_K,), jnp.int32),
        pltpu.VMEM((_ROWS, _NX), jnp.float32),
        pltpu.VMEM((_ROWS, _NX), jnp.float32),
        pltpu.VMEM((_ROWS, _NX), jnp.float32),
        pltpu.VMEM((_ROWS, _NX), jnp.float32),
        pltpu.SemaphoreType.DMA,
        pltpu.SemaphoreType.DMA,
        pltpu.SemaphoreType.DMA,
        pltpu.SemaphoreType.DMA,
        pltpu.SemaphoreType.DMA,
        pltpu.SemaphoreType.DMA,
    ],
)


def kernel(voxel_coords, record_len, pillar_features):
    del record_len
    cb = voxel_coords[:, 0].astype(jnp.int32)
    cy = voxel_coords[:, 2].astype(jnp.int32)
    cx = voxel_coords[:, 3].astype(jnp.int32)
    pad = _PPAD - cb.shape[0]
    cb = jnp.concatenate([cb, jnp.full((pad,), _CAV, jnp.int32)])
    cy = jnp.concatenate([cy, jnp.zeros((pad,), jnp.int32)])
    cx = jnp.concatenate([cx, jnp.zeros((pad,), jnp.int32)])
    tab = _feature_table(pillar_features).reshape(_F * _PTAB)
    out, _ = _sc_scatter(cb, cy, cx, tab)
    return out
